# Initial kernel scaffold; baseline (speedup 1.0000x reference)
import jax
import jax.numpy as jnp
from jax.experimental import pallas as pl


def kernel(x_lig, x_poc, edge_feat, edge_index, prj_src_W, prj_src_b, prj_dst_W, prj_dst_b, prj_edge_W, prj_edge_b, w_src_W, w_src_b, w_dst_W, w_dst_b, w_edge_W, w_edge_b, att_a, att_W, att_b, fc1_W, fc1_b, fc_a, bn_g, bn_b, bn_m, bn_v, fc2_W, fc2_b):
    raise NotImplementedError("write your pallas kernel here")


# speedup vs baseline: 3.8444x; 3.8444x over previous
import functools

import jax
import jax.numpy as jnp
from jax import lax
from jax.experimental import pallas as pl
from jax.experimental.pallas import tpu as pltpu
from jax.experimental.pallas import tpu_sc as plsc

N_NODE = 10000
E = 320000
D = 128
NC = 2
NS = 16
NT = NC * NS
EPT = E // NT
BLK = 80
NBLK = EPT // BLK
GRP = BLK // 16
SPAD = 10240
NEG = -3.0e38

_mesh = plsc.VectorSubcoreMesh(core_axis_name="c", subcore_axis_name="s")


def _proj2_body(x_ref, w1_ref, b1_ref, w2_ref, b2_ref, o1_ref, o2_ref):
    x = x_ref[...]
    o1_ref[...] = (
        jnp.dot(x, w1_ref[...], preferred_element_type=jnp.float32) + b1_ref[...]
    )
    o2_ref[...] = (
        jnp.dot(x, w2_ref[...], preferred_element_type=jnp.float32) + b2_ref[...]
    )


def _proj2(x, w1, b1, w2, b2, blk):
    n, k = x.shape
    grid = n // blk
    return pl.pallas_call(
        _proj2_body,
        grid=(grid,),
        in_specs=[
            pl.BlockSpec((blk, k), lambda i: (i, 0)),
            pl.BlockSpec((k, D), lambda i: (0, 0)),
            pl.BlockSpec((1, D), lambda i: (0, 0)),
            pl.BlockSpec((k, D), lambda i: (0, 0)),
            pl.BlockSpec((1, D), lambda i: (0, 0)),
        ],
        out_specs=[
            pl.BlockSpec((blk, D), lambda i: (i, 0)),
            pl.BlockSpec((blk, D), lambda i: (i, 0)),
        ],
        out_shape=[jax.ShapeDtypeStruct((n, D), jnp.float32)] * 2,
    )(x, w1, b1.reshape(1, D), w2, b2.reshape(1, D))


def _head_body(p_ref, w1_ref, b1_ref, a_ref, g_ref, be_ref, m_ref, v_ref,
               w2_ref, b2_ref, o_ref):
    ro = jnp.sum(p_ref[...], axis=0, keepdims=True)
    h = jnp.dot(ro, w1_ref[...], preferred_element_type=jnp.float32) + b1_ref[...]
    h = jnp.where(h >= 0.0, h, a_ref[...] * h)
    h = (h - m_ref[...]) / jnp.sqrt(v_ref[...] + 1e-5) * g_ref[...] + be_ref[...]
    o_ref[...] = (
        jnp.dot(h, w2_ref[...], preferred_element_type=jnp.float32) + b2_ref[...]
    )


def _head(part, fc1_W, fc1_b, fc_a, bn_g, bn_b, bn_m, bn_v, fc2_W, fc2_b):
    h2 = D // 2
    return pl.pallas_call(
        _head_body,
        out_shape=jax.ShapeDtypeStruct((1, 1), jnp.float32),
    )(
        part,
        fc1_W,
        fc1_b.reshape(1, h2),
        jnp.full((1, h2), fc_a, jnp.float32),
        bn_g.reshape(1, h2),
        bn_b.reshape(1, h2),
        bn_m.reshape(1, h2),
        bn_v.reshape(1, h2),
        fc2_W,
        fc2_b.reshape(1, 1),
    )


def _k1_body(hs, hd, e1, src_a, dst_a, attw, attb, alpha,
             wf_out, mx_out,
             idx_s, idx_d, rows_s, rows_d, rows_e, wf_blk, t256,
             attw_v, attb_v, alpha_v, mx_v, sem):
    cid = lax.axis_index("c")
    sid = lax.axis_index("s")
    wid = sid * NC + cid
    base = wid * EPT
    pltpu.sync_copy(attw, attw_v)
    pltpu.sync_copy(attb, attb_v)
    pltpu.sync_copy(alpha, alpha_v)
    al = alpha_v[...]
    ab = attb_v[...]
    aw_t = [attw_v[pl.ds(t * 16, 16)] for t in range(8)]
    scat_base = lax.iota(jnp.int32, 16) * 16

    def blk_body(b, mx):
        off = base + b * BLK
        pltpu.sync_copy(src_a.at[pl.ds(off, BLK)], idx_s)
        pltpu.sync_copy(dst_a.at[pl.ds(off, BLK)], idx_d)
        pltpu.async_copy(hs.at[idx_s], rows_s, sem).wait()
        pltpu.async_copy(hd.at[idx_d], rows_d, sem).wait()
        pltpu.sync_copy(e1.at[pl.ds(off, BLK)], rows_e)
        for g in range(GRP):

            def edge_body(j, carry):
                ei = g * 16 + j
                acc = jnp.zeros((16,), jnp.float32)
                for t in range(8):
                    vs = rows_s[ei, pl.ds(t * 16, 16)]
                    vd = rows_d[ei, pl.ds(t * 16, 16)]
                    ve = rows_e[ei, pl.ds(t * 16, 16)]
                    w = vs + vd + ve
                    p = jnp.where(w >= 0.0, w, al * w)
                    acc = acc + aw_t[t] * p
                plsc.store_scatter(t256, [scat_base + j], acc)
                return carry

            lax.fori_loop(0, 16, edge_body, 0)
            wfv = t256[pl.ds(0, 16)]
            for c in range(1, 16):
                wfv = wfv + t256[pl.ds(c * 16, 16)]
            wfv = wfv + ab
            wf_blk[pl.ds(g * 16, 16)] = wfv
            mx = jnp.maximum(mx, wfv)
        pltpu.sync_copy(wf_blk, wf_out.at[pl.ds(off, BLK)])
        return mx

    mx = lax.fori_loop(0, NBLK, blk_body, jnp.full((16,), NEG, jnp.float32))
    mx_v[...] = mx
    pltpu.sync_copy(mx_v, mx_out.at[wid])


def _k1(hs, hd, e1, src, dst, attw, attb16, alpha16):
    return pl.kernel(
        _k1_body,
        out_type=[
            jax.ShapeDtypeStruct((E,), jnp.float32),
            jax.ShapeDtypeStruct((NT, 16), jnp.float32),
        ],
        mesh=_mesh,
        compiler_params=pltpu.CompilerParams(needs_layout_passes=False),
        scratch_types=[
            pltpu.VMEM((BLK,), jnp.int32),
            pltpu.VMEM((BLK,), jnp.int32),
            pltpu.VMEM((BLK, D), jnp.float32),
            pltpu.VMEM((BLK, D), jnp.float32),
            pltpu.VMEM((BLK, D), jnp.float32),
            pltpu.VMEM((BLK,), jnp.float32),
            pltpu.VMEM((256,), jnp.float32),
            pltpu.VMEM((D,), jnp.float32),
            pltpu.VMEM((16,), jnp.float32),
            pltpu.VMEM((16,), jnp.float32),
            pltpu.VMEM((16,), jnp.float32),
            pltpu.SemaphoreType.DMA,
        ],
    )(hs, hd, e1, src, dst, attw, attb16, alpha16)


def _k2_body(wf, dst_a, mx_a,
             z_out, s_out,
             wf_t, z_t, idxb_v, mx_v, zero_v, s_sh, sem):
    cid = lax.axis_index("c")
    sid = lax.axis_index("s")
    wid = sid * NC + cid
    base = wid * EPT
    pltpu.sync_copy(mx_a, mx_v)
    m = jnp.full((16,), NEG, jnp.float32)
    for i in range(NT):
        m = jnp.maximum(m, mx_v[i, ...])
    mv = jnp.full((16,), jnp.max(m), jnp.float32)

    pltpu.sync_copy(wf.at[pl.ds(base, EPT)], wf_t)

    def zstep(i, carry):
        z_t[pl.ds(i * 16, 16)] = jnp.exp(wf_t[pl.ds(i * 16, 16)] - mv)
        return carry

    lax.fori_loop(0, EPT // 16, zstep, 0)
    pltpu.sync_copy(z_t, z_out.at[pl.ds(base, EPT)])

    @pl.when(sid == 0)
    def _zero():
        def zz(i, carry):
            zero_v[pl.ds(i * 16, 16)] = jnp.zeros((16,), jnp.float32)
            return carry

        lax.fori_loop(0, SPAD // 16, zz, 0)
        pltpu.sync_copy(zero_v, s_sh)

    plsc.subcore_barrier()

    def scat(b, carry):
        pltpu.sync_copy(dst_a.at[pl.ds(base + b * BLK, BLK)], idxb_v)
        pltpu.sync_copy(z_t.at[pl.ds(b * BLK, BLK)], s_sh.at[idxb_v], add=True)
        return carry

    lax.fori_loop(0, NBLK, scat, 0)
    plsc.subcore_barrier()

    @pl.when(sid == 0)
    def _flush():
        pltpu.sync_copy(s_sh, s_out.at[cid])


def _k2(wf, dst, mx):
    return pl.kernel(
        _k2_body,
        out_type=[
            jax.ShapeDtypeStruct((E,), jnp.float32),
            jax.ShapeDtypeStruct((NC, SPAD), jnp.float32),
        ],
        mesh=_mesh,
        compiler_params=pltpu.CompilerParams(needs_layout_passes=False),
        scratch_types=[
            pltpu.VMEM((EPT,), jnp.float32),
            pltpu.VMEM((EPT,), jnp.float32),
            pltpu.VMEM((BLK,), jnp.int32),
            pltpu.VMEM((NT, 16), jnp.float32),
            pltpu.VMEM((SPAD,), jnp.float32),
            pltpu.VMEM_SHARED((SPAD,), jnp.float32),
            pltpu.SemaphoreType.DMA,
        ],
    )(wf, dst, mx)


def _k3_body(z_a, src_a, dst_a, hs2, hd2, e2, s_a,
             part_out,
             idx_s, idx_d, z_blk, a_blk, rows_s, rows_d, rows_e,
             gs_v, gsb_v, acc_v, sem):
    cid = lax.axis_index("c")
    sid = lax.axis_index("s")
    wid = sid * NC + cid
    base = wid * EPT
    pltpu.sync_copy(s_a.at[0], gs_v)
    pltpu.sync_copy(s_a.at[1], gsb_v)

    def addg(i, carry):
        gs_v[pl.ds(i * 16, 16)] = gs_v[pl.ds(i * 16, 16)] + gsb_v[pl.ds(i * 16, 16)]
        return carry

    lax.fori_loop(0, SPAD // 16, addg, 0)

    def blk_body(b, accs):
        off = base + b * BLK
        pltpu.sync_copy(src_a.at[pl.ds(off, BLK)], idx_s)
        pltpu.sync_copy(dst_a.at[pl.ds(off, BLK)], idx_d)
        pltpu.sync_copy(z_a.at[pl.ds(off, BLK)], z_blk)
        pltpu.async_copy(hs2.at[idx_s], rows_s, sem).wait()
        pltpu.async_copy(hd2.at[idx_d], rows_d, sem).wait()
        pltpu.sync_copy(e2.at[pl.ds(off, BLK)], rows_e)
        for g in range(GRP):
            zv = z_blk[pl.ds(g * 16, 16)]
            dv = idx_d[pl.ds(g * 16, 16)]
            sv = plsc.load_gather(gs_v, [dv])
            a_blk[pl.ds(g * 16, 16)] = zv / sv

        def edge_body(j, accs2):
            aj = plsc.load_gather(a_blk, [jnp.full((16,), j, jnp.int32)])
            out = []
            for t in range(8):
                vs = rows_s[j, pl.ds(t * 16, 16)]
                vd = rows_d[j, pl.ds(t * 16, 16)]
                ve = rows_e[j, pl.ds(t * 16, 16)]
                out.append(accs2[t] + aj * vs * vd * ve)
            return tuple(out)

        return lax.fori_loop(0, BLK, edge_body, accs)

    accs = lax.fori_loop(
        0, NBLK, blk_body, tuple(jnp.zeros((16,), jnp.float32) for _ in range(8))
    )
    for t in range(8):
        acc_v[pl.ds(t * 16, 16)] = accs[t]
    pltpu.sync_copy(acc_v, part_out.at[wid])


def _k3(z, src, dst, hs2, hd2, e2, s_all):
    return pl.kernel(
        _k3_body,
        out_type=jax.ShapeDtypeStruct((NT, D), jnp.float32),
        mesh=_mesh,
        compiler_params=pltpu.CompilerParams(needs_layout_passes=False),
        scratch_types=[
            pltpu.VMEM((BLK,), jnp.int32),
            pltpu.VMEM((BLK,), jnp.int32),
            pltpu.VMEM((BLK,), jnp.float32),
            pltpu.VMEM((BLK,), jnp.float32),
            pltpu.VMEM((BLK, D), jnp.float32),
            pltpu.VMEM((BLK, D), jnp.float32),
            pltpu.VMEM((BLK, D), jnp.float32),
            pltpu.VMEM((SPAD,), jnp.float32),
            pltpu.VMEM((SPAD,), jnp.float32),
            pltpu.VMEM((D,), jnp.float32),
            pltpu.SemaphoreType.DMA,
        ],
    )(z, src, dst, hs2, hd2, e2, s_all)


def kernel(x_lig, x_poc, edge_feat, edge_index,
           prj_src_W, prj_src_b, prj_dst_W, prj_dst_b, prj_edge_W, prj_edge_b,
           w_src_W, w_src_b, w_dst_W, w_dst_b, w_edge_W, w_edge_b,
           att_a, att_W, att_b,
           fc1_W, fc1_b, fc_a, bn_g, bn_b, bn_m, bn_v, fc2_W, fc2_b):
    src = edge_index[0]
    dst = edge_index[1]

    hs, hs2 = _proj2(x_lig, prj_src_W, prj_src_b, w_src_W, w_src_b, 2000)
    hd, hd2 = _proj2(x_poc, prj_dst_W, prj_dst_b, w_dst_W, w_dst_b, 2000)
    e1, e2 = _proj2(edge_feat, prj_edge_W, prj_edge_b, w_edge_W, w_edge_b, 4000)

    attw = att_W[:, 0]
    attb16 = jnp.full((16,), att_b[0], jnp.float32)
    alpha16 = jnp.full((16,), att_a, jnp.float32)
    wf, mx = _k1(hs, hd, e1, src, dst, attw, attb16, alpha16)
    z, s_all = _k2(wf, dst, mx)
    part = _k3(z, src, dst, hs2, hd2, e2, s_all)
    return _head(part, fc1_W, fc1_b, fc_a, bn_g, bn_b, bn_m, bn_v, fc2_W, fc2_b)


# speedup vs baseline: 8.2807x; 2.1540x over previous
import functools

import jax
import jax.numpy as jnp
from jax import lax
from jax.experimental import pallas as pl
from jax.experimental.pallas import tpu as pltpu
from jax.experimental.pallas import tpu_sc as plsc

N_NODE = 10000
E = 320000
D = 128
NC = 2
NS = 16
NT = NC * NS
EPT = E // NT
BLK = 80
NBLK = EPT // BLK
GRP = BLK // 16
SPAD = 10240
NEG = -3.0e38

_mesh = plsc.VectorSubcoreMesh(core_axis_name="c", subcore_axis_name="s")


def _proj2_body(x_ref, w1_ref, b1_ref, w2_ref, b2_ref, o1_ref, o2_ref):
    x = x_ref[...]
    o1_ref[...] = (
        jnp.dot(x, w1_ref[...], preferred_element_type=jnp.float32) + b1_ref[...]
    )
    o2_ref[...] = (
        jnp.dot(x, w2_ref[...], preferred_element_type=jnp.float32) + b2_ref[...]
    )


def _proj2(x, w1, b1, w2, b2, blk):
    n, k = x.shape
    grid = n // blk
    return pl.pallas_call(
        _proj2_body,
        grid=(grid,),
        in_specs=[
            pl.BlockSpec((blk, k), lambda i: (i, 0)),
            pl.BlockSpec((k, D), lambda i: (0, 0)),
            pl.BlockSpec((1, D), lambda i: (0, 0)),
            pl.BlockSpec((k, D), lambda i: (0, 0)),
            pl.BlockSpec((1, D), lambda i: (0, 0)),
        ],
        out_specs=[
            pl.BlockSpec((blk, D), lambda i: (i, 0)),
            pl.BlockSpec((blk, D), lambda i: (i, 0)),
        ],
        out_shape=[jax.ShapeDtypeStruct((n, D), jnp.float32)] * 2,
    )(x, w1, b1.reshape(1, D), w2, b2.reshape(1, D))


def _head_body(p_ref, w1_ref, b1_ref, a_ref, g_ref, be_ref, m_ref, v_ref,
               w2_ref, b2_ref, o_ref):
    ro = jnp.sum(p_ref[...], axis=0, keepdims=True)
    h = jnp.dot(ro, w1_ref[...], preferred_element_type=jnp.float32) + b1_ref[...]
    h = jnp.where(h >= 0.0, h, a_ref[...] * h)
    h = (h - m_ref[...]) / jnp.sqrt(v_ref[...] + 1e-5) * g_ref[...] + be_ref[...]
    o_ref[...] = (
        jnp.dot(h, w2_ref[...], preferred_element_type=jnp.float32) + b2_ref[...]
    )


def _head(part, fc1_W, fc1_b, fc_a, bn_g, bn_b, bn_m, bn_v, fc2_W, fc2_b):
    h2 = D // 2
    return pl.pallas_call(
        _head_body,
        out_shape=jax.ShapeDtypeStruct((1, 1), jnp.float32),
    )(
        part,
        fc1_W,
        fc1_b.reshape(1, h2),
        jnp.full((1, h2), fc_a, jnp.float32),
        bn_g.reshape(1, h2),
        bn_b.reshape(1, h2),
        bn_m.reshape(1, h2),
        bn_v.reshape(1, h2),
        fc2_W,
        fc2_b.reshape(1, 1),
    )


def _k1_body(hs, hd, e1, src_a, dst_a, attw, attb, alpha,
             wf_out, mx_out,
             src_t, dst_t, rows_sA, rows_dA, rows_eA, rows_sB, rows_dB,
             rows_eB, wf_blk, t256, attw_v, attb_v, alpha_v, mx_v,
             semA, semB):
    cid = lax.axis_index("c")
    sid = lax.axis_index("s")
    wid = sid * NC + cid
    base = wid * EPT
    pltpu.sync_copy(attw, attw_v)
    pltpu.sync_copy(attb, attb_v)
    pltpu.sync_copy(alpha, alpha_v)
    pltpu.sync_copy(src_a.at[pl.ds(base, EPT)], src_t)
    pltpu.sync_copy(dst_a.at[pl.ds(base, EPT)], dst_t)
    al = alpha_v[...]
    ab = attb_v[...]
    aw_t = [attw_v[pl.ds(t * 16, 16)] for t in range(8)]
    scat_base = lax.iota(jnp.int32, 16) * 16

    def issue(b, rows_s, rows_d, rows_e, sem):
        pltpu.async_copy(hs.at[src_t.at[pl.ds(b * BLK, BLK)]], rows_s, sem)
        pltpu.async_copy(hd.at[dst_t.at[pl.ds(b * BLK, BLK)]], rows_d, sem)
        pltpu.async_copy(e1.at[pl.ds(base + b * BLK, BLK)], rows_e, sem)

    def wait(b, rows_s, rows_d, rows_e, sem):
        pltpu.make_async_copy(hs.at[src_t.at[pl.ds(b * BLK, BLK)]], rows_s, sem).wait()
        pltpu.make_async_copy(hd.at[dst_t.at[pl.ds(b * BLK, BLK)]], rows_d, sem).wait()
        pltpu.make_async_copy(e1.at[pl.ds(base + b * BLK, BLK)], rows_e, sem).wait()

    def compute(b, rows_s, rows_d, rows_e, mx):
        for g in range(GRP):

            def edge_body(j, carry):
                ei = g * 16 + j
                acc = jnp.zeros((16,), jnp.float32)
                for t in range(8):
                    vs = rows_s[ei, pl.ds(t * 16, 16)]
                    vd = rows_d[ei, pl.ds(t * 16, 16)]
                    ve = rows_e[ei, pl.ds(t * 16, 16)]
                    w = vs + vd + ve
                    p = jnp.where(w >= 0.0, w, al * w)
                    acc = acc + aw_t[t] * p
                plsc.store_scatter(t256, [scat_base + j], acc)
                return carry

            lax.fori_loop(0, 16, edge_body, 0)
            wfv = t256[pl.ds(0, 16)]
            for c in range(1, 16):
                wfv = wfv + t256[pl.ds(c * 16, 16)]
            wfv = wfv + ab
            wf_blk[pl.ds(g * 16, 16)] = wfv
            mx = jnp.maximum(mx, wfv)
        pltpu.sync_copy(wf_blk, wf_out.at[pl.ds(base + b * BLK, BLK)])
        return mx

    issue(0, rows_sA, rows_dA, rows_eA, semA)

    def pair(i, mx):
        b0 = 2 * i
        issue(b0 + 1, rows_sB, rows_dB, rows_eB, semB)
        wait(b0, rows_sA, rows_dA, rows_eA, semA)
        mx = compute(b0, rows_sA, rows_dA, rows_eA, mx)
        issue(b0 + 2, rows_sA, rows_dA, rows_eA, semA)
        wait(b0 + 1, rows_sB, rows_dB, rows_eB, semB)
        return compute(b0 + 1, rows_sB, rows_dB, rows_eB, mx)

    mx = lax.fori_loop(0, NBLK // 2, pair, jnp.full((16,), NEG, jnp.float32))
    wait(NBLK - 1, rows_sA, rows_dA, rows_eA, semA)
    mx = compute(NBLK - 1, rows_sA, rows_dA, rows_eA, mx)
    mx_v[...] = mx
    pltpu.sync_copy(mx_v, mx_out.at[wid])


def _k1(hs, hd, e1, src, dst, attw, attb16, alpha16):
    return pl.kernel(
        _k1_body,
        out_type=[
            jax.ShapeDtypeStruct((E,), jnp.float32),
            jax.ShapeDtypeStruct((NT, 16), jnp.float32),
        ],
        mesh=_mesh,
        compiler_params=pltpu.CompilerParams(needs_layout_passes=False),
        scratch_types=[
            pltpu.VMEM((EPT,), jnp.int32),
            pltpu.VMEM((EPT,), jnp.int32),
            pltpu.VMEM((BLK, D), jnp.float32),
            pltpu.VMEM((BLK, D), jnp.float32),
            pltpu.VMEM((BLK, D), jnp.float32),
            pltpu.VMEM((BLK, D), jnp.float32),
            pltpu.VMEM((BLK, D), jnp.float32),
            pltpu.VMEM((BLK, D), jnp.float32),
            pltpu.VMEM((BLK,), jnp.float32),
            pltpu.VMEM((256,), jnp.float32),
            pltpu.VMEM((D,), jnp.float32),
            pltpu.VMEM((16,), jnp.float32),
            pltpu.VMEM((16,), jnp.float32),
            pltpu.VMEM((16,), jnp.float32),
            pltpu.SemaphoreType.DMA,
            pltpu.SemaphoreType.DMA,
        ],
    )(hs, hd, e1, src, dst, attw, attb16, alpha16)


def _k2_body(wf, dst_a, mx_a,
             z_out, s_out,
             wf_t, z_t, idxb_v, mx_v, zero_v, s_sh, sem):
    cid = lax.axis_index("c")
    sid = lax.axis_index("s")
    wid = sid * NC + cid
    base = wid * EPT
    pltpu.sync_copy(mx_a, mx_v)
    m = jnp.full((16,), NEG, jnp.float32)
    for i in range(NT):
        m = jnp.maximum(m, mx_v[i, ...])
    mv = jnp.full((16,), jnp.max(m), jnp.float32)

    pltpu.sync_copy(wf.at[pl.ds(base, EPT)], wf_t)

    def zstep(i, carry):
        z_t[pl.ds(i * 16, 16)] = jnp.exp(wf_t[pl.ds(i * 16, 16)] - mv)
        return carry

    lax.fori_loop(0, EPT // 16, zstep, 0)
    pltpu.sync_copy(z_t, z_out.at[pl.ds(base, EPT)])

    @pl.when(sid == 0)
    def _zero():
        def zz(i, carry):
            zero_v[pl.ds(i * 16, 16)] = jnp.zeros((16,), jnp.float32)
            return carry

        lax.fori_loop(0, SPAD // 16, zz, 0)
        pltpu.sync_copy(zero_v, s_sh)

    plsc.subcore_barrier()

    def scat(b, carry):
        pltpu.sync_copy(dst_a.at[pl.ds(base + b * BLK, BLK)], idxb_v)
        pltpu.sync_copy(z_t.at[pl.ds(b * BLK, BLK)], s_sh.at[idxb_v], add=True)
        return carry

    lax.fori_loop(0, NBLK, scat, 0)
    plsc.subcore_barrier()

    @pl.when(sid == 0)
    def _flush():
        pltpu.sync_copy(s_sh, s_out.at[cid])


def _k2(wf, dst, mx):
    return pl.kernel(
        _k2_body,
        out_type=[
            jax.ShapeDtypeStruct((E,), jnp.float32),
            jax.ShapeDtypeStruct((NC, SPAD), jnp.float32),
        ],
        mesh=_mesh,
        compiler_params=pltpu.CompilerParams(needs_layout_passes=False),
        scratch_types=[
            pltpu.VMEM((EPT,), jnp.float32),
            pltpu.VMEM((EPT,), jnp.float32),
            pltpu.VMEM((BLK,), jnp.int32),
            pltpu.VMEM((NT, 16), jnp.float32),
            pltpu.VMEM((SPAD,), jnp.float32),
            pltpu.VMEM_SHARED((SPAD,), jnp.float32),
            pltpu.SemaphoreType.DMA,
        ],
    )(wf, dst, mx)


def _k3_body(z_a, src_a, dst_a, hs2, hd2, e2, s_a,
             part_out,
             src_t, dst_t, z_t, a_blk, rows_sA, rows_dA, rows_eA,
             rows_sB, rows_dB, rows_eB, gs_v, gsb_v, acc_v, semA, semB):
    cid = lax.axis_index("c")
    sid = lax.axis_index("s")
    wid = sid * NC + cid
    base = wid * EPT
    pltpu.sync_copy(s_a.at[0], gs_v)
    pltpu.sync_copy(s_a.at[1], gsb_v)
    pltpu.sync_copy(src_a.at[pl.ds(base, EPT)], src_t)
    pltpu.sync_copy(dst_a.at[pl.ds(base, EPT)], dst_t)
    pltpu.sync_copy(z_a.at[pl.ds(base, EPT)], z_t)

    def addg(i, carry):
        gs_v[pl.ds(i * 16, 16)] = gs_v[pl.ds(i * 16, 16)] + gsb_v[pl.ds(i * 16, 16)]
        return carry

    lax.fori_loop(0, SPAD // 16, addg, 0)

    def issue(b, rows_s, rows_d, rows_e, sem):
        pltpu.async_copy(hs2.at[src_t.at[pl.ds(b * BLK, BLK)]], rows_s, sem)
        pltpu.async_copy(hd2.at[dst_t.at[pl.ds(b * BLK, BLK)]], rows_d, sem)
        pltpu.async_copy(e2.at[pl.ds(base + b * BLK, BLK)], rows_e, sem)

    def wait(b, rows_s, rows_d, rows_e, sem):
        pltpu.make_async_copy(hs2.at[src_t.at[pl.ds(b * BLK, BLK)]], rows_s, sem).wait()
        pltpu.make_async_copy(hd2.at[dst_t.at[pl.ds(b * BLK, BLK)]], rows_d, sem).wait()
        pltpu.make_async_copy(e2.at[pl.ds(base + b * BLK, BLK)], rows_e, sem).wait()

    def compute(b, rows_s, rows_d, rows_e, accs):
        for g in range(GRP):
            zv = z_t[pl.ds(b * BLK + g * 16, 16)]
            dv = dst_t[pl.ds(b * BLK + g * 16, 16)]
            sv = plsc.load_gather(gs_v, [dv])
            a_blk[pl.ds(g * 16, 16)] = zv / sv

        def edge_body(j, accs2):
            aj = plsc.load_gather(a_blk, [jnp.full((16,), j, jnp.int32)])
            out = []
            for t in range(8):
                vs = rows_s[j, pl.ds(t * 16, 16)]
                vd = rows_d[j, pl.ds(t * 16, 16)]
                ve = rows_e[j, pl.ds(t * 16, 16)]
                out.append(accs2[t] + aj * vs * vd * ve)
            return tuple(out)

        return lax.fori_loop(0, BLK, edge_body, accs)

    issue(0, rows_sA, rows_dA, rows_eA, semA)

    def pair(i, accs):
        b0 = 2 * i
        issue(b0 + 1, rows_sB, rows_dB, rows_eB, semB)
        wait(b0, rows_sA, rows_dA, rows_eA, semA)
        accs = compute(b0, rows_sA, rows_dA, rows_eA, accs)
        issue(b0 + 2, rows_sA, rows_dA, rows_eA, semA)
        wait(b0 + 1, rows_sB, rows_dB, rows_eB, semB)
        return compute(b0 + 1, rows_sB, rows_dB, rows_eB, accs)

    accs = lax.fori_loop(
        0, NBLK // 2, pair,
        tuple(jnp.zeros((16,), jnp.float32) for _ in range(8)),
    )
    wait(NBLK - 1, rows_sA, rows_dA, rows_eA, semA)
    accs = compute(NBLK - 1, rows_sA, rows_dA, rows_eA, accs)
    for t in range(8):
        acc_v[pl.ds(t * 16, 16)] = accs[t]
    pltpu.sync_copy(acc_v, part_out.at[wid])


def _k3(z, src, dst, hs2, hd2, e2, s_all):
    return pl.kernel(
        _k3_body,
        out_type=jax.ShapeDtypeStruct((NT, D), jnp.float32),
        mesh=_mesh,
        compiler_params=pltpu.CompilerParams(needs_layout_passes=False),
        scratch_types=[
            pltpu.VMEM((EPT,), jnp.int32),
            pltpu.VMEM((EPT,), jnp.int32),
            pltpu.VMEM((EPT,), jnp.float32),
            pltpu.VMEM((BLK,), jnp.float32),
            pltpu.VMEM((BLK, D), jnp.float32),
            pltpu.VMEM((BLK, D), jnp.float32),
            pltpu.VMEM((BLK, D), jnp.float32),
            pltpu.VMEM((BLK, D), jnp.float32),
            pltpu.VMEM((BLK, D), jnp.float32),
            pltpu.VMEM((BLK, D), jnp.float32),
            pltpu.VMEM((SPAD,), jnp.float32),
            pltpu.VMEM((SPAD,), jnp.float32),
            pltpu.VMEM((D,), jnp.float32),
            pltpu.SemaphoreType.DMA,
            pltpu.SemaphoreType.DMA,
        ],
    )(z, src, dst, hs2, hd2, e2, s_all)


def kernel(x_lig, x_poc, edge_feat, edge_index,
           prj_src_W, prj_src_b, prj_dst_W, prj_dst_b, prj_edge_W, prj_edge_b,
           w_src_W, w_src_b, w_dst_W, w_dst_b, w_edge_W, w_edge_b,
           att_a, att_W, att_b,
           fc1_W, fc1_b, fc_a, bn_g, bn_b, bn_m, bn_v, fc2_W, fc2_b):
    src = edge_index[0]
    dst = edge_index[1]

    hs, hs2 = _proj2(x_lig, prj_src_W, prj_src_b, w_src_W, w_src_b, 2000)
    hd, hd2 = _proj2(x_poc, prj_dst_W, prj_dst_b, w_dst_W, w_dst_b, 2000)
    e1, e2 = _proj2(edge_feat, prj_edge_W, prj_edge_b, w_edge_W, w_edge_b, 4000)

    attw = att_W[:, 0]
    attb16 = jnp.full((16,), att_b[0], jnp.float32)
    alpha16 = jnp.full((16,), att_a, jnp.float32)
    wf, mx = _k1(hs, hd, e1, src, dst, attw, attb16, alpha16)
    z, s_all = _k2(wf, dst, mx)
    part = _k3(z, src, dst, hs2, hd2, e2, s_all)
    return _head(part, fc1_W, fc1_b, fc_a, bn_g, bn_b, bn_m, bn_v, fc2_W, fc2_b)
